# baseline (device time: 116284 ns/iter reference)
import jax
import jax.numpy as jnp
from jax import lax
from jax.experimental import pallas as pl
from jax.experimental.pallas import tpu as pltpu

B, S, H, D = 4, 256, 16, 64
SCALE = D ** -0.5
HALF = S // 2


def _dot(a, b, dims):
    return lax.dot_general(
        a, b, (dims, ((), ())), preferred_element_type=jnp.float32,
        precision=lax.Precision.DEFAULT,
    )


def _body(q_ref, k_ref, v_ref, o_ref, kr_ref, vr_ref, m_sc, d_sc,
          z_send, z_recv, x_send, x_recv):
    my_x = lax.axis_index("x")
    my_y = lax.axis_index("y")
    my_z = lax.axis_index("z")
    peer_z = (my_x, my_y, 1 - my_z)
    peer_x = (1 - my_x, my_y, my_z)
    off = my_x * HALF
    off_o = (1 - my_x) * HALF

    barrier_sem = pltpu.get_barrier_semaphore()
    for nbr in (peer_z, peer_x):
        pl.semaphore_signal(
            barrier_sem, inc=1, device_id=nbr,
            device_id_type=pl.DeviceIdType.MESH,
        )
    pl.semaphore_wait(barrier_sem, 2)

    pairs = [(k_ref, kr_ref), (v_ref, vr_ref)]
    chunks = [(b, t) for b in range(B) for t in range(2)]

    z_rdmas = []
    for i, (b, t) in enumerate(chunks):
        src, dst = pairs[t]
        r = pltpu.make_async_remote_copy(
            src_ref=src.at[b, pl.ds(off, HALF)],
            dst_ref=dst.at[b, pl.ds(off, HALF)],
            send_sem=z_send.at[i],
            recv_sem=z_recv.at[i],
            device_id=peer_z,
            device_id_type=pl.DeviceIdType.MESH,
        )
        r.start()
        z_rdmas.append(r)

    x_fwds = []
    x_rcvs = []
    for i, (b, t) in enumerate(chunks):
        buf = pairs[t][1]
        x_fwds.append(pltpu.make_async_remote_copy(
            src_ref=buf.at[b, pl.ds(off, HALF)],
            dst_ref=buf.at[b, pl.ds(off, HALF)],
            send_sem=x_send.at[i],
            recv_sem=x_recv.at[i],
            device_id=peer_x,
            device_id_type=pl.DeviceIdType.MESH,
        ))
        x_rcvs.append(pltpu.make_async_remote_copy(
            src_ref=buf.at[b, pl.ds(off_o, HALF)],
            dst_ref=buf.at[b, pl.ds(off_o, HALF)],
            send_sem=z_send.at[i],
            recv_sem=x_recv.at[i],
            device_id=peer_x,
            device_id_type=pl.DeviceIdType.MESH,
        ))

    def local_pass(b, hs):
        for h in hs:
            col = slice(h * D, (h + 1) * D)
            q = q_ref[b, :, col]
            s = _dot(q, k_ref[b, :, col], ((1,), (1,))) * SCALE
            m = jnp.max(s, axis=1, keepdims=True)
            p = jnp.exp(s - m)
            d = jnp.sum(p, axis=1, keepdims=True)
            o_ref[b, :, col] = _dot(p, v_ref[b, :, col], ((1,), (0,)))
            m_sc[b, :, h:h + 1] = m
            d_sc[b, :, h:h + 1] = d

    def remote_pass(b):
        for h in range(H):
            col = slice(h * D, (h + 1) * D)
            q = q_ref[b, :, col]
            s = _dot(q, kr_ref[b, :, col], ((1,), (1,))) * SCALE
            m_r = jnp.max(s, axis=1, keepdims=True)
            p = jnp.exp(s - m_r)
            d_r = jnp.sum(p, axis=1, keepdims=True)
            o_r = _dot(p, vr_ref[b, :, col], ((1,), (0,)))
            m_l = m_sc[b, :, h:h + 1]
            d_l = d_sc[b, :, h:h + 1]
            m = jnp.maximum(m_l, m_r)
            a_l = jnp.exp(m_l - m)
            a_r = jnp.exp(m_r - m)
            o_ref[b, :, col] = (
                o_ref[b, :, col] * a_l + o_r * a_r
            ) / (d_l * a_l + d_r * a_r)

    for i, (b, t) in enumerate(chunks):
        z_rdmas[i].wait_recv()
        x_fwds[i].start()
        half = i % 2
        local_pass(i // 2, range(half * (H // 2), (half + 1) * (H // 2)))

    for b in range(B):
        x_rcvs[2 * b].wait_recv()
        x_rcvs[2 * b + 1].wait_recv()
        remote_pass(b)

    for i in range(len(chunks)):
        z_rdmas[i].wait_send()
        x_fwds[i].wait_send()


def kernel(Q, K, V):
    q2 = Q.reshape(B, S, H * D)
    k2 = K.reshape(B, S, H * D)
    v2 = V.reshape(B, S, H * D)
    out = pl.pallas_call(
        _body,
        out_shape=jax.ShapeDtypeStruct((B, S, H * D), jnp.float32),
        in_specs=[
            pl.BlockSpec(memory_space=pltpu.VMEM),
            pl.BlockSpec(memory_space=pltpu.VMEM),
            pl.BlockSpec(memory_space=pltpu.VMEM),
        ],
        out_specs=pl.BlockSpec(memory_space=pltpu.VMEM),
        scratch_shapes=[
            pltpu.VMEM((B, S, H * D), jnp.float32),
            pltpu.VMEM((B, S, H * D), jnp.float32),
            pltpu.VMEM((B, S, H), jnp.float32),
            pltpu.VMEM((B, S, H), jnp.float32),
            pltpu.SemaphoreType.DMA((8,)),
            pltpu.SemaphoreType.DMA((8,)),
            pltpu.SemaphoreType.DMA((8,)),
            pltpu.SemaphoreType.DMA((8,)),
        ],
        compiler_params=pltpu.CompilerParams(
            collective_id=0, vmem_limit_bytes=64 * 1024 * 1024
        ),
    )(q2, k2, v2)
    return out.reshape(B, S, H, D)
